# baseline (speedup 1.0000x reference)
import functools

import jax
import jax.numpy as jnp
import numpy as np
from jax import lax
from jax.experimental import pallas as pl
from jax.experimental.pallas import tpu as pltpu
from jax.experimental.pallas import tpu_sc as plsc

_L = 16


def _ceil_to(x, m):
    return -(-x // m) * m


@functools.lru_cache(maxsize=None)
def _build(M, K, V):
    n_out = M * V
    n_pad = _ceil_to(n_out, _L)
    n_idx_pad = _ceil_to(2 * V, _L)
    n_ivf = M * K * V
    scale = 1.0 / M
    mesh = plsc.VectorSubcoreMesh(core_axis_name="c", subcore_axis_name="s")

    def _ivec(val):
        return jnp.full((_L,), val, jnp.int32)

    @functools.partial(
        pl.kernel,
        mesh=mesh,
        compiler_params=pltpu.CompilerParams(needs_layout_passes=False),
        out_type=jax.ShapeDtypeStruct((n_pad,), jnp.float32),
        scratch_types=[
            pltpu.VMEM((n_ivf,), jnp.float32),
            pltpu.VMEM((n_idx_pad,), jnp.int32),
            pltpu.VMEM((n_pad,), jnp.float32),
            pltpu.VMEM((n_pad,), jnp.float32),
        ],
    )
    def sc_kernel(ivf_hbm, idx_hbm, wout_hbm, out_hbm, ivf_v, idx_v, wout_v, out_v):
        wid = lax.axis_index("s") * 2 + lax.axis_index("c")

        @pl.when(wid == 0)
        def _():
            pltpu.sync_copy(ivf_hbm, ivf_v)
            pltpu.sync_copy(idx_hbm, idx_v)
            pltpu.sync_copy(wout_hbm, wout_v)
            scale_vec = jnp.full((_L,), scale, jnp.float32)
            v_vec = _ivec(V)
            lane = lax.iota(jnp.int32, _L)
            for chunk in range(n_pad // _L):
                e = jnp.minimum(lane + _ivec(chunk * _L), _ivec(n_out - 1))
                m = lax.div(e, v_vec)
                v = e - m * v_vec
                base = m * _ivec(K * V) + v
                i0 = plsc.load_gather(idx_v, [v])
                i1 = plsc.load_gather(idx_v, [v + v_vec])
                a = plsc.load_gather(ivf_v, [base + i0 * v_vec])
                b = plsc.load_gather(ivf_v, [base + i1 * v_vec])
                w = wout_v[pl.ds(chunk * _L, _L)]
                out_v[pl.ds(chunk * _L, _L)] = scale_vec * a * b * w
            pltpu.sync_copy(out_v, out_hbm)

    return sc_kernel


def kernel(num_M, num_VN, IVF, VN_index, wout):
    M, K, V = IVF.shape
    n_out = M * V
    n_pad = _ceil_to(n_out, _L)
    n_idx_pad = _ceil_to(2 * V, _L)
    ivf_flat = IVF.reshape(M * K * V).astype(jnp.float32)
    idx_flat = (
        jnp.zeros((n_idx_pad,), jnp.int32)
        .at[: 2 * V]
        .set(VN_index.astype(jnp.int32).reshape(2 * V))
    )
    wout_flat = (
        jnp.zeros((n_pad,), jnp.float32).at[:n_out].set(wout.reshape(n_out))
    )
    out = _build(M, K, V)(ivf_flat, idx_flat, wout_flat)
    return out[:n_out].reshape(M, V)


# speedup vs baseline: 1.1504x; 1.1504x over previous
import functools

import jax
import jax.numpy as jnp
from jax import lax
from jax.experimental import pallas as pl
from jax.experimental.pallas import tpu as pltpu
from jax.experimental.pallas import tpu_sc as plsc

_L = 16


def _ceil_to(x, m):
    return -(-x // m) * m


@functools.lru_cache(maxsize=None)
def _build(M, K, V):
    n_out = M * V
    n_pad = _ceil_to(n_out, _L)
    idx_off = M * K * V
    wout_off = idx_off + _ceil_to(2 * V, 8)
    n_packed = _ceil_to(wout_off + n_out, _L)
    scale = 1.0 / M
    mesh = plsc.VectorSubcoreMesh(
        core_axis_name="c", subcore_axis_name="s", num_cores=1, num_subcores=1
    )

    def _ivec(val):
        return jnp.full((_L,), val, jnp.int32)

    @functools.partial(
        pl.kernel,
        mesh=mesh,
        compiler_params=pltpu.CompilerParams(needs_layout_passes=False),
        out_type=jax.ShapeDtypeStruct((n_pad,), jnp.float32),
        scratch_types=[
            pltpu.VMEM((n_packed,), jnp.float32),
            pltpu.VMEM((n_pad,), jnp.float32),
        ],
    )
    def sc_kernel(packed_hbm, out_hbm, packed_v, out_v):
        pltpu.sync_copy(packed_hbm, packed_v)
        scale_vec = jnp.full((_L,), scale, jnp.float32)
        v_vec = _ivec(V)
        lane = lax.iota(jnp.int32, _L)
        for chunk in range(n_pad // _L):
            e = jnp.minimum(lane + _ivec(chunk * _L), _ivec(n_out - 1))
            m = lax.div(e, v_vec)
            v = e - m * v_vec
            i0 = plsc.bitcast(
                plsc.load_gather(packed_v, [v + _ivec(idx_off)]), jnp.int32
            )
            i1 = plsc.bitcast(
                plsc.load_gather(packed_v, [v + _ivec(idx_off + V)]), jnp.int32
            )
            base = m * _ivec(K * V) + v
            a = plsc.load_gather(packed_v, [base + i0 * v_vec])
            b = plsc.load_gather(packed_v, [base + i1 * v_vec])
            w = packed_v[pl.ds(wout_off + chunk * _L, _L)]
            out_v[pl.ds(chunk * _L, _L)] = scale_vec * a * b * w
        pltpu.sync_copy(out_v, out_hbm)

    return sc_kernel


def kernel(num_M, num_VN, IVF, VN_index, wout):
    M, K, V = IVF.shape
    n_out = M * V
    idx_off = M * K * V
    wout_off = idx_off + _ceil_to(2 * V, 8)
    n_packed = _ceil_to(wout_off + n_out, _L)
    idx_f32 = lax.bitcast_convert_type(
        VN_index.astype(jnp.int32).reshape(2 * V), jnp.float32
    )
    packed = jnp.zeros((n_packed,), jnp.float32)
    packed = packed.at[:idx_off].set(IVF.reshape(idx_off).astype(jnp.float32))
    packed = packed.at[idx_off : idx_off + 2 * V].set(idx_f32)
    packed = packed.at[wout_off : wout_off + n_out].set(
        wout.reshape(n_out).astype(jnp.float32)
    )
    out = _build(M, K, V)(packed)
    return out[:n_out].reshape(M, V)


# speedup vs baseline: 1.1514x; 1.0008x over previous
import functools

import jax
import jax.numpy as jnp
from jax import lax
from jax.experimental import pallas as pl
from jax.experimental.pallas import tpu as pltpu
from jax.experimental.pallas import tpu_sc as plsc

_L = 16


def _ceil_to(x, m):
    return -(-x // m) * m


@functools.lru_cache(maxsize=None)
def _build(M, K, V):
    n_out = M * V
    n_pad = _ceil_to(n_out, _L)
    idx_off = M * K * V
    wout_off = idx_off + _ceil_to(2 * V, 8)
    n_packed = _ceil_to(wout_off + n_out, _L)
    scale = 1.0 / M
    mesh = plsc.VectorSubcoreMesh(
        core_axis_name="c", subcore_axis_name="s", num_cores=1, num_subcores=1
    )

    def _ivec(val):
        return jnp.full((_L,), val, jnp.int32)

    @functools.partial(
        pl.kernel,
        mesh=mesh,
        compiler_params=pltpu.CompilerParams(
            needs_layout_passes=False,
            disable_bounds_checks=True,
            disable_semaphore_checks=True,
            skip_device_barrier=True,
        ),
        out_type=jax.ShapeDtypeStruct((n_pad,), jnp.float32),
        scratch_types=[
            pltpu.VMEM((n_packed,), jnp.float32),
            pltpu.VMEM((n_pad,), jnp.float32),
        ],
    )
    def sc_kernel(packed_hbm, out_hbm, packed_v, out_v):
        pltpu.sync_copy(packed_hbm, packed_v)
        scale_vec = jnp.full((_L,), scale, jnp.float32)
        v_vec = _ivec(V)
        lane = lax.iota(jnp.int32, _L)
        for chunk in range(n_pad // _L):
            e = jnp.minimum(lane + _ivec(chunk * _L), _ivec(n_out - 1))
            m = lax.div(e, v_vec)
            v = e - m * v_vec
            i0 = plsc.bitcast(
                plsc.load_gather(packed_v, [v + _ivec(idx_off)]), jnp.int32
            )
            i1 = plsc.bitcast(
                plsc.load_gather(packed_v, [v + _ivec(idx_off + V)]), jnp.int32
            )
            base = m * _ivec(K * V) + v
            a = plsc.load_gather(packed_v, [base + i0 * v_vec])
            b = plsc.load_gather(packed_v, [base + i1 * v_vec])
            w = packed_v[pl.ds(wout_off + chunk * _L, _L)]
            out_v[pl.ds(chunk * _L, _L)] = scale_vec * a * b * w
        pltpu.sync_copy(out_v, out_hbm)

    return sc_kernel


def kernel(num_M, num_VN, IVF, VN_index, wout):
    M, K, V = IVF.shape
    n_out = M * V
    idx_off = M * K * V
    wout_off = idx_off + _ceil_to(2 * V, 8)
    n_packed = _ceil_to(wout_off + n_out, _L)
    idx_f32 = lax.bitcast_convert_type(
        VN_index.astype(jnp.int32).reshape(2 * V), jnp.float32
    )
    packed = jnp.zeros((n_packed,), jnp.float32)
    packed = packed.at[:idx_off].set(IVF.reshape(idx_off).astype(jnp.float32))
    packed = packed.at[idx_off : idx_off + 2 * V].set(idx_f32)
    packed = packed.at[wout_off : wout_off + n_out].set(
        wout.reshape(n_out).astype(jnp.float32)
    )
    out = _build(M, K, V)(packed)
    return out[:n_out].reshape(M, V)


# speedup vs baseline: 1.1577x; 1.0055x over previous
import functools

import jax
import jax.numpy as jnp
from jax import lax
from jax.experimental import pallas as pl
from jax.experimental.pallas import tpu as pltpu
from jax.experimental.pallas import tpu_sc as plsc

_L = 16


def _ceil_to(x, m):
    return -(-x // m) * m


@functools.lru_cache(maxsize=None)
def _build(M, K, V):
    n_out = M * V
    n_pad = _ceil_to(n_out, _L)
    wout_off = M * K * V
    n_data = _ceil_to(wout_off + n_out, _L)
    n_idx = _ceil_to(2 * V, _L)
    scale = 1.0 / M
    mesh = plsc.ScalarSubcoreMesh(axis_name="c", num_cores=1)

    @functools.partial(
        pl.kernel,
        mesh=mesh,
        compiler_params=pltpu.CompilerParams(
            needs_layout_passes=False,
            disable_bounds_checks=True,
            disable_semaphore_checks=True,
            skip_device_barrier=True,
        ),
        out_type=jax.ShapeDtypeStruct((n_pad,), jnp.float32),
        scratch_types=[
            pltpu.SMEM((n_data,), jnp.float32),
            pltpu.SMEM((n_idx,), jnp.int32),
            pltpu.SMEM((n_pad,), jnp.float32),
        ],
    )
    def scs_kernel(data_hbm, idx_hbm, out_hbm, data_s, idx_s, out_s):
        pltpu.sync_copy(data_hbm, data_s)
        pltpu.sync_copy(idx_hbm, idx_s)

        def body(i, carry):
            m = lax.div(i, V)
            v = i - m * V
            i0 = idx_s[v]
            i1 = idx_s[v + V]
            base = m * (K * V) + v
            a = data_s[base + i0 * V]
            b = data_s[base + i1 * V]
            w = data_s[wout_off + i]
            out_s[i] = scale * a * b * w
            return carry

        lax.fori_loop(0, n_out, body, 0)
        pltpu.sync_copy(out_s, out_hbm)

    return scs_kernel


def kernel(num_M, num_VN, IVF, VN_index, wout):
    M, K, V = IVF.shape
    n_out = M * V
    wout_off = M * K * V
    n_data = _ceil_to(wout_off + n_out, _L)
    n_idx = _ceil_to(2 * V, _L)
    data = jnp.zeros((n_data,), jnp.float32)
    data = data.at[:wout_off].set(IVF.reshape(wout_off).astype(jnp.float32))
    data = data.at[wout_off : wout_off + n_out].set(
        wout.reshape(n_out).astype(jnp.float32)
    )
    idx = (
        jnp.zeros((n_idx,), jnp.int32)
        .at[: 2 * V]
        .set(VN_index.astype(jnp.int32).reshape(2 * V))
    )
    out = _build(M, K, V)(data, idx)
    return out[:n_out].reshape(M, V)
